# baseline (speedup 1.0000x reference)
import jax
import jax.numpy as jnp
from jax import lax
from jax.experimental import pallas as pl

_G = 4


def _build_k_kernel(ei_ref, k_ref):
    GV = k_ref.shape[0]
    V = GV // _G
    E = ei_ref.shape[1]
    ei = ei_ref[...]
    row = ei[0:1, :]
    col = ei[1:2, :]
    ids = lax.broadcasted_iota(jnp.int32, (V, E), 0)
    C = (ids == col).astype(jnp.float32)
    R = (ids == row).astype(jnp.float32)
    deg = jnp.sum(C, axis=1, keepdims=True) + 1.0
    dinv = lax.rsqrt(deg)
    dr = lax.dot_general(dinv, R, (((0,), (0,)), ((), ())),
                         preferred_element_type=jnp.float32)
    dc = lax.dot_general(dinv, C, (((0,), (0,)), ((), ())),
                         preferred_element_type=jnp.float32)
    Cn = C * (dr * dc)
    A = lax.dot_general(Cn, R, (((1,), (1,)), ((), ())),
                        preferred_element_type=jnp.float32)
    eye = (lax.broadcasted_iota(jnp.int32, (V, V), 0)
           == lax.broadcasted_iota(jnp.int32, (V, V), 1)).astype(jnp.float32)
    A = A + eye * (dinv * dinv)
    p = lax.broadcasted_iota(jnp.int32, (GV, GV), 0)
    q = lax.broadcasted_iota(jnp.int32, (GV, GV), 1)
    same_block = ((p // V) == (q // V)).astype(jnp.float32)
    Pv = (lax.broadcasted_iota(jnp.int32, (GV, V), 0) % V
          == lax.broadcasted_iota(jnp.int32, (GV, V), 1)).astype(jnp.float32)
    t1 = lax.dot_general(Pv, A, (((1,), (1,)), ((), ())),
                         preferred_element_type=jnp.float32)
    at_big = lax.dot_general(t1, Pv, (((1,), (1,)), ((), ())),
                             preferred_element_type=jnp.float32)
    k_ref[...] = at_big * same_block


def _main_kernel(x_ref, w_ref, k_ref, b_ref, o_ref):
    _, O, T, V = o_ref.shape
    C = x_ref.shape[1]
    TG = T // _G
    GV = _G * V
    xm = x_ref[0].reshape(C, T * V)
    y = lax.dot_general(w_ref[...], xm, (((0,), (0,)), ((), ())),
                        preferred_element_type=jnp.float32)
    y3 = y.reshape(O, TG, GV) + jnp.zeros((1, 1, GV), jnp.float32)
    y2 = y3.reshape(O * TG, GV)
    u = lax.dot_general(y2, k_ref[...], (((1,), (0,)), ((), ())),
                        preferred_element_type=jnp.float32)
    u3 = u.reshape(O, TG, GV) + jnp.zeros((1, 1, GV), jnp.float32)
    um = u3.reshape(O, T * V) + b_ref[...]
    o_ref[0] = um.reshape(O, T, V)


def kernel(x, W, b, edge_index):
    n, c, t, v = x.shape
    o = W.shape[1]
    ei = edge_index.astype(jnp.int32)
    gv = _G * v

    K = pl.pallas_call(
        _build_k_kernel,
        out_shape=jax.ShapeDtypeStruct((gv, gv), jnp.float32),
    )(ei)

    b2 = b.reshape(o, 1)

    out = pl.pallas_call(
        _main_kernel,
        grid=(n,),
        in_specs=[
            pl.BlockSpec((1, c, t, v), lambda i: (i, 0, 0, 0)),
            pl.BlockSpec((c, o), lambda i: (0, 0)),
            pl.BlockSpec((gv, gv), lambda i: (0, 0)),
            pl.BlockSpec((o, 1), lambda i: (0, 0)),
        ],
        out_specs=pl.BlockSpec((1, o, t, v), lambda i: (i, 0, 0, 0)),
        out_shape=jax.ShapeDtypeStruct((n, o, t, v), jnp.float32),
    )(x, W, K, b2)
    return out


# speedup vs baseline: 1.2362x; 1.2362x over previous
import jax
import jax.numpy as jnp
from jax import lax
from jax.experimental import pallas as pl

_G = 4
_VP = 32


def _build_k_kernel(ei_ref, k_ref):
    GV = k_ref.shape[0]
    VP = GV // _G
    E = ei_ref.shape[1]
    ei = ei_ref[...]
    row = ei[0:1, :]
    col = ei[1:2, :]
    ids = lax.broadcasted_iota(jnp.int32, (VP, E), 0)
    C = (ids == col).astype(jnp.float32)
    R = (ids == row).astype(jnp.float32)
    deg = jnp.sum(C, axis=1, keepdims=True) + 1.0
    dinv = lax.rsqrt(deg)
    dr = lax.dot_general(dinv, R, (((0,), (0,)), ((), ())),
                         preferred_element_type=jnp.float32)
    dc = lax.dot_general(dinv, C, (((0,), (0,)), ((), ())),
                         preferred_element_type=jnp.float32)
    Cn = C * (dr * dc)
    A = lax.dot_general(Cn, R, (((1,), (1,)), ((), ())),
                        preferred_element_type=jnp.float32)
    eye = (lax.broadcasted_iota(jnp.int32, (VP, VP), 0)
           == lax.broadcasted_iota(jnp.int32, (VP, VP), 1)).astype(jnp.float32)
    A = A + eye * (dinv * dinv)
    p = lax.broadcasted_iota(jnp.int32, (GV, GV), 0)
    q = lax.broadcasted_iota(jnp.int32, (GV, GV), 1)
    same_block = ((p // VP) == (q // VP)).astype(jnp.float32)
    Pv = (lax.broadcasted_iota(jnp.int32, (GV, VP), 0) % VP
          == lax.broadcasted_iota(jnp.int32, (GV, VP), 1)).astype(jnp.float32)
    t1 = lax.dot_general(Pv, A, (((1,), (1,)), ((), ())),
                         preferred_element_type=jnp.float32)
    at_big = lax.dot_general(t1, Pv, (((1,), (1,)), ((), ())),
                             preferred_element_type=jnp.float32)
    k_ref[...] = at_big * same_block


def _main_kernel(x_ref, w_ref, k_ref, b_ref, o_ref):
    _, O, L = o_ref.shape
    GVP = k_ref.shape[0]
    y = lax.dot_general(w_ref[...], x_ref[0], (((0,), (0,)), ((), ())),
                        preferred_element_type=jnp.float32)
    K = k_ref[...]
    nslab = L // GVP
    slabs = []
    for g in range(nslab):
        ys = y[:, g * GVP:(g + 1) * GVP]
        slabs.append(lax.dot_general(ys, K, (((1,), (0,)), ((), ())),
                                     preferred_element_type=jnp.float32))
    u = jnp.concatenate(slabs, axis=1)
    o_ref[0] = u + b_ref[...]


def kernel(x, W, b, edge_index):
    n, c, t, v = x.shape
    o = W.shape[1]
    ei = edge_index.astype(jnp.int32)
    gvp = _G * _VP

    K = pl.pallas_call(
        _build_k_kernel,
        out_shape=jax.ShapeDtypeStruct((gvp, gvp), jnp.float32),
    )(ei)

    b2 = b.reshape(o, 1)
    xp = jnp.pad(x, ((0, 0), (0, 0), (0, 0), (0, _VP - v)))
    xp = xp.reshape(n, c, t * _VP)

    out = pl.pallas_call(
        _main_kernel,
        grid=(n,),
        in_specs=[
            pl.BlockSpec((1, c, t * _VP), lambda i: (i, 0, 0)),
            pl.BlockSpec((c, o), lambda i: (0, 0)),
            pl.BlockSpec((gvp, gvp), lambda i: (0, 0)),
            pl.BlockSpec((o, 1), lambda i: (0, 0)),
        ],
        out_specs=pl.BlockSpec((1, o, t * _VP), lambda i: (i, 0, 0)),
        out_shape=jax.ShapeDtypeStruct((n, o, t * _VP), jnp.float32),
    )(xp, W, K, b2)
    return out.reshape(n, o, t, _VP)[..., :v]


# speedup vs baseline: 1.7550x; 1.4196x over previous
import jax
import jax.numpy as jnp
from jax import lax
from jax.experimental import pallas as pl

_G = 4
_NB = 4


def _build_k_kernel(ei_ref, k_ref):
    GV = k_ref.shape[0]
    V = GV // _G
    E = ei_ref.shape[1]
    ei = ei_ref[...]
    row = ei[0:1, :]
    col = ei[1:2, :]
    ids = lax.broadcasted_iota(jnp.int32, (V, E), 0)
    C = (ids == col).astype(jnp.float32)
    R = (ids == row).astype(jnp.float32)
    deg = jnp.sum(C, axis=1, keepdims=True) + 1.0
    dinv = lax.rsqrt(deg)
    dr = lax.dot_general(dinv, R, (((0,), (0,)), ((), ())),
                         preferred_element_type=jnp.float32)
    dc = lax.dot_general(dinv, C, (((0,), (0,)), ((), ())),
                         preferred_element_type=jnp.float32)
    Cn = C * (dr * dc)
    A = lax.dot_general(Cn, R, (((1,), (1,)), ((), ())),
                        preferred_element_type=jnp.float32)
    eye = (lax.broadcasted_iota(jnp.int32, (V, V), 0)
           == lax.broadcasted_iota(jnp.int32, (V, V), 1)).astype(jnp.float32)
    A = A + eye * (dinv * dinv)
    p = lax.broadcasted_iota(jnp.int32, (GV, GV), 0)
    q = lax.broadcasted_iota(jnp.int32, (GV, GV), 1)
    same_block = ((p // V) == (q // V)).astype(jnp.float32)
    Pv = (lax.broadcasted_iota(jnp.int32, (GV, V), 0) % V
          == lax.broadcasted_iota(jnp.int32, (GV, V), 1)).astype(jnp.float32)
    t1 = lax.dot_general(Pv, A, (((1,), (1,)), ((), ())),
                         preferred_element_type=jnp.float32)
    at_big = lax.dot_general(t1, Pv, (((1,), (1,)), ((), ())),
                             preferred_element_type=jnp.float32)
    k_ref[...] = at_big * same_block


def _main_kernel(x_ref, w_ref, k_ref, b_ref, o_ref):
    NB, O, TG, GV = o_ref.shape
    K = k_ref[...]
    for b in range(NB):
        y = lax.dot_general(w_ref[...], x_ref[b], (((0,), (0,)), ((), ())),
                            preferred_element_type=jnp.float32)
        y3 = y.reshape(O, TG, GV) + jnp.zeros((1, 1, GV), jnp.float32)
        y2 = y3.reshape(O * TG, GV)
        u = lax.dot_general(y2, K, (((1,), (0,)), ((), ())),
                            preferred_element_type=jnp.float32)
        o_ref[b] = u.reshape(O, TG, GV) + b_ref[...].reshape(O, 1, 1)


def kernel(x, W, b, edge_index):
    n, c, t, v = x.shape
    o = W.shape[1]
    ei = edge_index.astype(jnp.int32)
    gv = _G * v
    tg = t // _G

    K = pl.pallas_call(
        _build_k_kernel,
        out_shape=jax.ShapeDtypeStruct((gv, gv), jnp.float32),
    )(ei)

    b2 = b.reshape(o, 1)
    x2 = x.reshape(n, c, t * v)

    out = pl.pallas_call(
        _main_kernel,
        grid=(n // _NB,),
        in_specs=[
            pl.BlockSpec((_NB, c, t * v), lambda i: (i, 0, 0)),
            pl.BlockSpec((c, o), lambda i: (0, 0)),
            pl.BlockSpec((gv, gv), lambda i: (0, 0)),
            pl.BlockSpec((o, 1), lambda i: (0, 0)),
        ],
        out_specs=pl.BlockSpec((_NB, o, tg, gv), lambda i: (i, 0, 0, 0)),
        out_shape=jax.ShapeDtypeStruct((n, o, tg, gv), jnp.float32),
    )(x2, W, K, b2)
    return out.reshape(n, o, t, v)


# speedup vs baseline: 1.7803x; 1.0144x over previous
import jax
import jax.numpy as jnp
from jax import lax
from jax.experimental import pallas as pl

_G = 4
_NB = 8


def _build_k_kernel(ei_ref, k_ref):
    GV = k_ref.shape[0]
    V = GV // _G
    E = ei_ref.shape[1]
    ei = ei_ref[...]
    row = ei[0:1, :]
    col = ei[1:2, :]
    ids = lax.broadcasted_iota(jnp.int32, (V, E), 0)
    C = (ids == col).astype(jnp.float32)
    R = (ids == row).astype(jnp.float32)
    deg = jnp.sum(C, axis=1, keepdims=True) + 1.0
    dinv = lax.rsqrt(deg)
    dr = lax.dot_general(dinv, R, (((0,), (0,)), ((), ())),
                         preferred_element_type=jnp.float32)
    dc = lax.dot_general(dinv, C, (((0,), (0,)), ((), ())),
                         preferred_element_type=jnp.float32)
    Cn = C * (dr * dc)
    A = lax.dot_general(Cn, R, (((1,), (1,)), ((), ())),
                        preferred_element_type=jnp.float32)
    eye = (lax.broadcasted_iota(jnp.int32, (V, V), 0)
           == lax.broadcasted_iota(jnp.int32, (V, V), 1)).astype(jnp.float32)
    A = A + eye * (dinv * dinv)
    p = lax.broadcasted_iota(jnp.int32, (GV, GV), 0)
    q = lax.broadcasted_iota(jnp.int32, (GV, GV), 1)
    same_block = ((p // V) == (q // V)).astype(jnp.float32)
    Pv = (lax.broadcasted_iota(jnp.int32, (GV, V), 0) % V
          == lax.broadcasted_iota(jnp.int32, (GV, V), 1)).astype(jnp.float32)
    t1 = lax.dot_general(Pv, A, (((1,), (1,)), ((), ())),
                         preferred_element_type=jnp.float32)
    at_big = lax.dot_general(t1, Pv, (((1,), (1,)), ((), ())),
                             preferred_element_type=jnp.float32)
    k_ref[...] = at_big * same_block


def _main_kernel(x_ref, w_ref, k_ref, b_ref, o_ref):
    NB, O, TG, GV = o_ref.shape
    K = k_ref[...]
    for b in range(NB):
        y = lax.dot_general(w_ref[...], x_ref[b], (((0,), (0,)), ((), ())),
                            preferred_element_type=jnp.float32)
        y3 = y.reshape(O, TG, GV) + jnp.zeros((1, 1, GV), jnp.float32)
        y2 = y3.reshape(O * TG, GV)
        u = lax.dot_general(y2, K, (((1,), (0,)), ((), ())),
                            preferred_element_type=jnp.float32)
        o_ref[b] = u.reshape(O, TG, GV) + b_ref[...].reshape(O, 1, 1)


def kernel(x, W, b, edge_index):
    n, c, t, v = x.shape
    o = W.shape[1]
    ei = edge_index.astype(jnp.int32)
    gv = _G * v
    tg = t // _G

    K = pl.pallas_call(
        _build_k_kernel,
        out_shape=jax.ShapeDtypeStruct((gv, gv), jnp.float32),
    )(ei)

    b2 = b.reshape(o, 1)
    x2 = x.reshape(n, c, t * v)

    out = pl.pallas_call(
        _main_kernel,
        grid=(n // _NB,),
        in_specs=[
            pl.BlockSpec((_NB, c, t * v), lambda i: (i, 0, 0)),
            pl.BlockSpec((c, o), lambda i: (0, 0)),
            pl.BlockSpec((gv, gv), lambda i: (0, 0)),
            pl.BlockSpec((o, 1), lambda i: (0, 0)),
        ],
        out_specs=pl.BlockSpec((_NB, o, tg, gv), lambda i: (i, 0, 0, 0)),
        out_shape=jax.ShapeDtypeStruct((n, o, tg, gv), jnp.float32),
    )(x2, W, K, b2)
    return out.reshape(n, o, t, v)
